# baseline (speedup 1.0000x reference)
import functools

import jax
import jax.numpy as jnp
from jax import lax
from jax.experimental import pallas as pl
from jax.experimental.pallas import tpu as pltpu
from jax.experimental.pallas import tpu_sc as plsc


WIDE = 128
_NC = 2
_NS = 16
_GCHUNK = 1000


def _sc_gather(tr, tcb, ridx, cidx):
    e = ridx.shape[0]
    wide = tr.shape[1]
    nw = _NC * _NS
    per = e // nw
    nch = per // _GCHUNK
    mesh = plsc.VectorSubcoreMesh(core_axis_name="c", subcore_axis_name="s")

    def body(tr_hbm, tc_hbm, ri_hbm, ci_hbm, gr_hbm, gc_hbm, idx_v, rows_v, sem):
        wid = lax.axis_index("s") * _NC + lax.axis_index("c")

        def step(ch, carry):
            base = wid * per + ch * _GCHUNK
            pltpu.sync_copy(ri_hbm.at[pl.ds(base, _GCHUNK)], idx_v)
            pltpu.async_copy(tr_hbm.at[idx_v], rows_v, sem).wait()
            pltpu.sync_copy(rows_v, gr_hbm.at[pl.ds(base, _GCHUNK)])
            pltpu.sync_copy(ci_hbm.at[pl.ds(base, _GCHUNK)], idx_v)
            pltpu.async_copy(tc_hbm.at[idx_v], rows_v, sem).wait()
            pltpu.sync_copy(rows_v, gc_hbm.at[pl.ds(base, _GCHUNK)])
            return carry

        lax.fori_loop(0, nch, step, 0)

    f = pl.kernel(
        body,
        out_type=[jax.ShapeDtypeStruct((e, wide), jnp.float32),
                  jax.ShapeDtypeStruct((e, wide), jnp.float32)],
        mesh=mesh,
        scratch_types=[
            pltpu.VMEM((_GCHUNK,), jnp.int32),
            pltpu.VMEM((_GCHUNK, wide), jnp.float32),
            pltpu.SemaphoreType.DMA,
        ],
    )
    return f(tr, tcb, ridx, cidx)


_SCH = 400


_NB = 6


def _sc_scatter(eout, ridx, nseg):
    e = ridx.shape[0]
    wide = eout.shape[1]
    base = (nseg + _NB - 1) // _NB
    r = (base - 96 + 127) // 128 * 128 + 96
    rt = r + 32
    stripe = rt // _NS
    per = e // _NS
    nch = per // _SCH
    mesh = plsc.VectorSubcoreMesh(core_axis_name="c", subcore_axis_name="s")
    f = pl.kernel(
        _make_scatter_body(e, r, rt, stripe, per, nch, wide),
        out_type=jax.ShapeDtypeStruct((_NB, rt, wide), jnp.float32),
        mesh=mesh,
        scratch_types=[
            pltpu.VMEM((_SCH,), jnp.int32),
            pltpu.VMEM((_SCH,), jnp.int32),
            pltpu.VMEM((_SCH, wide), jnp.float32),
            pltpu.VMEM_SHARED((rt, wide), jnp.float32),
        ],
    )
    zeros = jnp.zeros((stripe, wide), jnp.float32)
    return f(eout, ridx, zeros)


def _make_scatter_body(e, r, rt, stripe, per, nch, wide):
    def body(eout_hbm, ridx_hbm, zeros_hbm, tab_hbm, idx_v, lidx_v, val_v,
             spmem):
        c = lax.axis_index("c")
        s = lax.axis_index("s")

        for ph in range(_NB // 2):
            b = c * (_NB // 2) + ph
            base_row = b * r
            pltpu.sync_copy(zeros_hbm, spmem.at[pl.ds(s * stripe, stripe)])
            plsc.subcore_barrier()

            def step(ch, carry):
                base = s * per + ch * _SCH
                pltpu.sync_copy(ridx_hbm.at[pl.ds(base, _SCH)], idx_v)
                pltpu.sync_copy(eout_hbm.at[pl.ds(base, _SCH)], val_v)
                for j in range(_SCH // 16):
                    v = idx_v[pl.ds(j * 16, 16)]
                    l = v - base_row
                    oob = (l < 0) | (l >= r)
                    lidx_v[pl.ds(j * 16, 16)] = jnp.where(oob, r, l)
                pltpu.sync_copy(val_v, spmem.at[lidx_v], add=True)
                return carry

            lax.fori_loop(0, nch, step, 0)
            plsc.subcore_barrier()
            pltpu.sync_copy(spmem.at[pl.ds(s * stripe, stripe)],
                            tab_hbm.at[b, pl.ds(s * stripe, stripe)])
            plsc.subcore_barrier()

    return body


def _silu(v):
    return v * jax.nn.sigmoid(v)


def _prologue_body(h_ref, wemb_ref, bemb_ref, temb_ref, out_ref):
    h0 = jnp.dot(h_ref[...], wemb_ref[...], preferred_element_type=jnp.float32)
    h0 = h0 + bemb_ref[...]
    out_ref[...] = h0[:, None, :] + temb_ref[...][None, :, :]


def _prologue(h, wemb, bemb, temb, n_blk):
    n, _ = h.shape
    npast, hdim = temb.shape
    grid = n // n_blk
    return pl.pallas_call(
        _prologue_body,
        grid=(grid,),
        in_specs=[
            pl.BlockSpec((n_blk, 1), lambda i: (i, 0)),
            pl.BlockSpec((1, hdim), lambda i: (0, 0)),
            pl.BlockSpec((1, hdim), lambda i: (0, 0)),
            pl.BlockSpec((npast, hdim), lambda i: (0, 0)),
        ],
        out_specs=pl.BlockSpec((n_blk, npast, hdim), lambda i: (i, 0, 0)),
        out_shape=jax.ShapeDtypeStruct((n, npast, hdim), jnp.float32),
    )(h, wemb, bemb, temb)


def _pq_body(hh_ref, xx_ref, wa_ref, wb_ref, tr_ref, tc_ref):
    hh = hh_ref[...]
    xx = xx_ref[...]
    b = hh.shape[0]
    pad = jnp.zeros((b, WIDE - wa_ref.shape[1] - 3), jnp.float32)
    p = jnp.dot(hh, wa_ref[...], preferred_element_type=jnp.float32)
    q = jnp.dot(hh, wb_ref[...], preferred_element_type=jnp.float32)
    tr_ref[...] = jnp.concatenate([p, xx, pad], axis=1)
    tc_ref[...] = jnp.concatenate([q, xx, pad], axis=1)


def _pq_tables(hh, xx, we1a, we1b, blk):
    nseg, hdim = hh.shape
    grid = nseg // blk
    return pl.pallas_call(
        _pq_body,
        grid=(grid,),
        in_specs=[
            pl.BlockSpec((blk, hdim), lambda i: (i, 0)),
            pl.BlockSpec((blk, 3), lambda i: (i, 0)),
            pl.BlockSpec((hdim, hdim), lambda i: (0, 0)),
            pl.BlockSpec((hdim, hdim), lambda i: (0, 0)),
        ],
        out_specs=[
            pl.BlockSpec((blk, WIDE), lambda i: (i, 0)),
            pl.BlockSpec((blk, WIDE), lambda i: (i, 0)),
        ],
        out_shape=[
            jax.ShapeDtypeStruct((nseg, WIDE), jnp.float32),
            jax.ShapeDtypeStruct((nseg, WIDE), jnp.float32),
        ],
    )(hh, xx, we1a, we1b)


def _edge_body(gr_ref, gc_ref, ea_ref, wear_ref, wr_ref, be1_ref, we2_ref,
               be2_ref, wc1_ref, bc1_ref, wc2_ref, out_ref):
    hdim = we2_ref.shape[0]
    gr = gr_ref[...]
    gc = gc_ref[...]
    p = gr[:, :hdim]
    q = gc[:, :hdim]
    diff = gr[:, hdim:hdim + 3] - gc[:, hdim:hdim + 3]
    radial = jnp.sum(diff * diff, axis=1, keepdims=True)
    pre = p + q + radial * wr_ref[...]
    pre = pre + jnp.dot(ea_ref[...], wear_ref[...],
                        preferred_element_type=jnp.float32)
    pre = pre + be1_ref[...]
    m = _silu(pre)
    m = _silu(jnp.dot(m, we2_ref[...], preferred_element_type=jnp.float32)
              + be2_ref[...])
    c1 = _silu(jnp.dot(m, wc1_ref[...], preferred_element_type=jnp.float32)
               + bc1_ref[...])
    cw = jnp.dot(c1, wc2_ref[...], preferred_element_type=jnp.float32)
    b = m.shape[0]
    ones = jnp.ones((b, 1), jnp.float32)
    pad = jnp.zeros((b, WIDE - hdim - 4), jnp.float32)
    out_ref[...] = jnp.concatenate([m, diff * cw, ones, pad], axis=1)


def _edge_compute(gr, gc, ea, wear, wr, be1, we2, be2, wc1, bc1, wc2, blk):
    e = gr.shape[0]
    hdim = we2.shape[0]
    grid = e // blk
    full = lambda i: (0, 0)
    return pl.pallas_call(
        _edge_body,
        grid=(grid,),
        in_specs=[
            pl.BlockSpec((blk, WIDE), lambda i: (i, 0)),
            pl.BlockSpec((blk, WIDE), lambda i: (i, 0)),
            pl.BlockSpec((blk, 3), lambda i: (i, 0)),
            pl.BlockSpec((3, hdim), full),
            pl.BlockSpec((1, hdim), full),
            pl.BlockSpec((1, hdim), full),
            pl.BlockSpec((hdim, hdim), full),
            pl.BlockSpec((1, hdim), full),
            pl.BlockSpec((hdim, hdim), full),
            pl.BlockSpec((1, hdim), full),
            pl.BlockSpec((hdim, 1), full),
        ],
        out_specs=pl.BlockSpec((blk, WIDE), lambda i: (i, 0)),
        out_shape=jax.ShapeDtypeStruct((e, WIDE), jnp.float32),
    )(gr, gc, ea, wear, wr, be1, we2, be2, wc1, bc1, wc2)


def _node_body(hh_ref, tab_ref, vv_ref, xx_ref, wv_ref, bv_ref, wn1a_ref,
               wn1b_ref, bn1_ref, wn2_ref, bn2_ref, wea_ref, web_ref,
               hh_out, vv_out, xx_out, tr_out, tc_out):
    hdim = wv_ref.shape[0]
    hh = hh_ref[...]
    tab = tab_ref[...]
    magg = tab[:, :hdim]
    cnt = jnp.maximum(tab[:, hdim + 3:hdim + 4], 1.0)
    agg = tab[:, hdim:hdim + 3] / cnt
    wv = jnp.dot(hh, wv_ref[...], preferred_element_type=jnp.float32) + bv_ref[...]
    vvn = wv * vv_ref[...] + agg
    xxn = xx_ref[...] + vvn
    pre = (jnp.dot(hh, wn1a_ref[...], preferred_element_type=jnp.float32)
           + jnp.dot(magg, wn1b_ref[...], preferred_element_type=jnp.float32)
           + bn1_ref[...])
    hhn = jnp.dot(_silu(pre), wn2_ref[...],
                  preferred_element_type=jnp.float32) + bn2_ref[...]
    hh_out[...] = hhn
    vv_out[...] = vvn
    xx_out[...] = xxn
    b = hh.shape[0]
    pad = jnp.zeros((b, WIDE - hdim - 3), jnp.float32)
    p = jnp.dot(hhn, wea_ref[...], preferred_element_type=jnp.float32)
    q = jnp.dot(hhn, web_ref[...], preferred_element_type=jnp.float32)
    tr_out[...] = jnp.concatenate([p, xxn, pad], axis=1)
    tc_out[...] = jnp.concatenate([q, xxn, pad], axis=1)


def _node_update(hh, tab, vv, xx, wv, bv, wn1a, wn1b, bn1, wn2, bn2,
                 wea, web, blk):
    nseg, hdim = hh.shape
    grid = nseg // blk
    full = lambda i: (0, 0)
    return pl.pallas_call(
        _node_body,
        grid=(grid,),
        in_specs=[
            pl.BlockSpec((blk, hdim), lambda i: (i, 0)),
            pl.BlockSpec((blk, WIDE), lambda i: (i, 0)),
            pl.BlockSpec((blk, 3), lambda i: (i, 0)),
            pl.BlockSpec((blk, 3), lambda i: (i, 0)),
            pl.BlockSpec((hdim, 1), full),
            pl.BlockSpec((1, 1), full),
            pl.BlockSpec((hdim, hdim), full),
            pl.BlockSpec((hdim, hdim), full),
            pl.BlockSpec((1, hdim), full),
            pl.BlockSpec((hdim, hdim), full),
            pl.BlockSpec((1, hdim), full),
            pl.BlockSpec((hdim, hdim), full),
            pl.BlockSpec((hdim, hdim), full),
        ],
        out_specs=[
            pl.BlockSpec((blk, hdim), lambda i: (i, 0)),
            pl.BlockSpec((blk, 3), lambda i: (i, 0)),
            pl.BlockSpec((blk, 3), lambda i: (i, 0)),
            pl.BlockSpec((blk, WIDE), lambda i: (i, 0)),
            pl.BlockSpec((blk, WIDE), lambda i: (i, 0)),
        ],
        out_shape=[
            jax.ShapeDtypeStruct((nseg, hdim), jnp.float32),
            jax.ShapeDtypeStruct((nseg, 3), jnp.float32),
            jax.ShapeDtypeStruct((nseg, 3), jnp.float32),
            jax.ShapeDtypeStruct((nseg, WIDE), jnp.float32),
            jax.ShapeDtypeStruct((nseg, WIDE), jnp.float32),
        ],
    )(hh, tab, vv, xx, wv, bv, wn1a, wn1b, bn1, wn2, bn2, wea, web)


def _logit_body(hh_ref, wa_ref, ba_ref, out_ref):
    out_ref[...] = jax.nn.sigmoid(
        jnp.dot(hh_ref[...], wa_ref[...], preferred_element_type=jnp.float32)
        + ba_ref[...])


def _logits(hh, wa, ba, blk):
    nseg, hdim = hh.shape
    grid = nseg // blk
    return pl.pallas_call(
        _logit_body,
        grid=(grid,),
        in_specs=[
            pl.BlockSpec((blk, hdim), lambda i: (i, 0)),
            pl.BlockSpec((hdim, 1), lambda i: (0, 0)),
            pl.BlockSpec((1, 1), lambda i: (0, 0)),
        ],
        out_specs=pl.BlockSpec((blk, 1), lambda i: (i, 0)),
        out_shape=jax.ShapeDtypeStruct((nseg, 1), jnp.float32),
    )(hh, wa, ba)


def _pool_body(w_ref, x_ref, out_ref):
    w = w_ref[...]
    w = w / (jnp.sum(w, axis=1, keepdims=True) + 1e-8)
    out_ref[...] = jnp.sum(w[:, :, None] * x_ref[...], axis=1)


def _pool(w3, x3, blk):
    n, npast = w3.shape
    grid = n // blk
    return pl.pallas_call(
        _pool_body,
        grid=(grid,),
        in_specs=[
            pl.BlockSpec((blk, npast), lambda i: (i, 0)),
            pl.BlockSpec((blk, npast, 3), lambda i: (i, 0, 0)),
        ],
        out_specs=pl.BlockSpec((blk, 3), lambda i: (i, 0)),
        out_shape=jax.ShapeDtypeStruct((n, 3), jnp.float32),
    )(w3, x3)


def kernel(h, x, edges, edge_attr, node_vel, cfg, W_emb, b_emb, time_emb,
           We1, be1, We2, be2, Wc1, bc1, Wc2, Wn1, bn1, Wn2, bn2, Wv, bv,
           Wa, ba):
    npast, n = x.shape[0], x.shape[1]
    nseg = npast * n
    hdim = W_emb.shape[1]
    nlayers = We1.shape[0]
    e = edges.shape[1]

    n_blk = 1000 if n % 1000 == 0 else n
    seg_blk = 2000 if nseg % 2000 == 0 else nseg
    e_blk = 2000 if e % 2000 == 0 else e

    hh3 = _prologue(h, W_emb, b_emb.reshape(1, hdim), time_emb, n_blk)
    hh = hh3.reshape(nseg, hdim)
    xx = jnp.transpose(x, (1, 0, 2)).reshape(nseg, 3)
    vv = node_vel.reshape(nseg, 3)
    row, col = edges[0], edges[1]

    tr, tc = _pq_tables(hh, xx, We1[0, :hdim], We1[0, hdim:2 * hdim], seg_blk)

    sc_ok = (e % (_NC * _NS * _GCHUNK) == 0) and (e % (_NS * _SCH) == 0)
    for i in range(nlayers):
        if sc_ok:
            gr, gc = _sc_gather(tr, tc, row, col)
        else:
            gr = jnp.take(tr, row, axis=0)
            gc = jnp.take(tc, col, axis=0)
        eout = _edge_compute(
            gr, gc, edge_attr,
            We1[i, 2 * hdim + 1:], We1[i, 2 * hdim:2 * hdim + 1],
            be1[i].reshape(1, hdim), We2[i], be2[i].reshape(1, hdim),
            Wc1[i], bc1[i].reshape(1, hdim), Wc2[i], e_blk)
        if sc_ok:
            tab4 = _sc_scatter(eout, row, nseg)
            rr = tab4.shape[1] - 32
            tab = jnp.concatenate(
                [tab4[b, :rr] for b in range(_NB)], axis=0)[:nseg]
        else:
            tab = jax.ops.segment_sum(eout, row, num_segments=nseg)
        if i + 1 < nlayers:
            wea = We1[i + 1, :hdim]
            web = We1[i + 1, hdim:2 * hdim]
        else:
            wea = jnp.zeros((hdim, hdim), jnp.float32)
            web = jnp.zeros((hdim, hdim), jnp.float32)
        hh, vv, xx, tr, tc = _node_update(
            hh, tab, vv, xx, Wv[i], bv[i].reshape(1, 1),
            Wn1[i, :hdim], Wn1[i, hdim:], bn1[i].reshape(1, hdim),
            Wn2[i], bn2[i].reshape(1, hdim), wea, web, seg_blk)

    w = _logits(hh, Wa, ba.reshape(1, 1), seg_blk)
    w3 = w.reshape(n, npast)
    x3 = xx.reshape(n, npast, 3)
    return _pool(w3, x3, n_blk)


# speedup vs baseline: 1.4750x; 1.4750x over previous
import functools

import jax
import jax.numpy as jnp
from jax import lax
from jax.experimental import pallas as pl
from jax.experimental.pallas import tpu as pltpu
from jax.experimental.pallas import tpu_sc as plsc


WIDE = 128
_NC = 2
_NS = 16
_GCHUNK = 1000


def _sc_gather(tr, tcb, ridx, cidx):
    e = ridx.shape[0]
    wide = tr.shape[1]
    nw = _NC * _NS
    per = e // nw
    nch = per // _GCHUNK
    mesh = plsc.VectorSubcoreMesh(core_axis_name="c", subcore_axis_name="s")

    def body(tr_hbm, tc_hbm, ri_hbm, ci_hbm, gr_hbm, gc_hbm, idx_v, rows_v, sem):
        wid = lax.axis_index("s") * _NC + lax.axis_index("c")

        def step(ch, carry):
            base = wid * per + ch * _GCHUNK
            pltpu.sync_copy(ri_hbm.at[pl.ds(base, _GCHUNK)], idx_v)
            pltpu.async_copy(tr_hbm.at[idx_v], rows_v, sem).wait()
            pltpu.sync_copy(rows_v, gr_hbm.at[pl.ds(base, _GCHUNK)])
            pltpu.sync_copy(ci_hbm.at[pl.ds(base, _GCHUNK)], idx_v)
            pltpu.async_copy(tc_hbm.at[idx_v], rows_v, sem).wait()
            pltpu.sync_copy(rows_v, gc_hbm.at[pl.ds(base, _GCHUNK)])
            return carry

        lax.fori_loop(0, nch, step, 0)

    f = pl.kernel(
        body,
        out_type=[jax.ShapeDtypeStruct((e, wide), jnp.float32),
                  jax.ShapeDtypeStruct((e, wide), jnp.float32)],
        mesh=mesh,
        scratch_types=[
            pltpu.VMEM((_GCHUNK,), jnp.int32),
            pltpu.VMEM((_GCHUNK, wide), jnp.float32),
            pltpu.SemaphoreType.DMA,
        ],
    )
    return f(tr, tcb, ridx, cidx)


_SCH = 400


_NB = 6


def _range_rows(nseg):
    base = (nseg + _NB - 1) // _NB
    return (base - 96 + 127) // 128 * 128 + 96


def _sc_scatter(eout, ridx, offs8, nseg):
    e = ridx.shape[0]
    wide = eout.shape[1]
    r = _range_rows(nseg)
    rt = r + 32
    stripe = rt // _NS
    mesh = plsc.VectorSubcoreMesh(core_axis_name="c", subcore_axis_name="s")
    f = pl.kernel(
        _make_scatter_body(e, r, rt, stripe, wide),
        out_type=jax.ShapeDtypeStruct((_NB, rt, wide), jnp.float32),
        mesh=mesh,
        scratch_types=[
            pltpu.VMEM((_SCH,), jnp.int32),
            pltpu.VMEM((_SCH,), jnp.int32),
            pltpu.VMEM((_SCH, wide), jnp.float32),
            pltpu.VMEM((16,), jnp.int32),
            pltpu.VMEM_SHARED((rt, wide), jnp.float32),
        ],
    )
    zeros = jnp.zeros((stripe, wide), jnp.float32)
    return f(eout, ridx, zeros, offs8)


def _make_scatter_body(e, r, rt, stripe, wide):
    nph = _NB // 2

    def body(eout_hbm, ridx_hbm, zeros_hbm, offs_hbm, tab_hbm,
             idx_v, lidx_v, val_v, offs_v, spmem):
        c = lax.axis_index("c")
        s = lax.axis_index("s")
        pltpu.sync_copy(offs_hbm, offs_v)
        ov = offs_v[pl.ds(0, 16)]

        for ph in range(nph):
            b = c * nph + ph
            base_row = b * r
            pltpu.sync_copy(zeros_hbm, spmem.at[pl.ds(s * stripe, stripe)])
            plsc.subcore_barrier()

            start0 = ov[ph] + c * (ov[nph + ph] - ov[ph])
            end = ov[ph + 1] + c * (ov[nph + ph + 1] - ov[ph + 1])
            start = start0 // 8 * 8
            nch = (end - start + _SCH - 1) // _SCH
            nch_s = (nch - s + _NS - 1) // _NS

            def step(k, carry):
                ch = s + k * _NS
                base = start + ch * _SCH
                basec = jnp.minimum(base, e - _SCH)
                pltpu.sync_copy(ridx_hbm.at[pl.ds(basec, _SCH)], idx_v)
                pltpu.sync_copy(eout_hbm.at[pl.ds(basec, _SCH)], val_v)
                for j in range(_SCH // 16):
                    pos = basec + j * 16 + lax.broadcasted_iota(
                        jnp.int32, (16,), 0)
                    v = idx_v[pl.ds(j * 16, 16)]
                    l = v - base_row
                    bad = (l < 0) | (l >= r) | (pos < base) | (pos >= end)
                    lidx_v[pl.ds(j * 16, 16)] = jnp.where(bad, r, l)
                pltpu.sync_copy(val_v, spmem.at[lidx_v], add=True)
                return carry

            lax.fori_loop(0, nch_s, step, 0)
            plsc.subcore_barrier()
            pltpu.sync_copy(spmem.at[pl.ds(s * stripe, stripe)],
                            tab_hbm.at[b, pl.ds(s * stripe, stripe)])
            plsc.subcore_barrier()

    return body


def _silu(v):
    return v * jax.nn.sigmoid(v)


def _prologue_body(h_ref, wemb_ref, bemb_ref, temb_ref, out_ref):
    h0 = jnp.dot(h_ref[...], wemb_ref[...], preferred_element_type=jnp.float32)
    h0 = h0 + bemb_ref[...]
    out_ref[...] = h0[:, None, :] + temb_ref[...][None, :, :]


def _prologue(h, wemb, bemb, temb, n_blk):
    n, _ = h.shape
    npast, hdim = temb.shape
    grid = n // n_blk
    return pl.pallas_call(
        _prologue_body,
        grid=(grid,),
        in_specs=[
            pl.BlockSpec((n_blk, 1), lambda i: (i, 0)),
            pl.BlockSpec((1, hdim), lambda i: (0, 0)),
            pl.BlockSpec((1, hdim), lambda i: (0, 0)),
            pl.BlockSpec((npast, hdim), lambda i: (0, 0)),
        ],
        out_specs=pl.BlockSpec((n_blk, npast, hdim), lambda i: (i, 0, 0)),
        out_shape=jax.ShapeDtypeStruct((n, npast, hdim), jnp.float32),
    )(h, wemb, bemb, temb)


def _pq_body(hh_ref, xx_ref, wa_ref, wb_ref, tr_ref, tc_ref):
    hh = hh_ref[...]
    xx = xx_ref[...]
    b = hh.shape[0]
    pad = jnp.zeros((b, WIDE - wa_ref.shape[1] - 3), jnp.float32)
    p = jnp.dot(hh, wa_ref[...], preferred_element_type=jnp.float32)
    q = jnp.dot(hh, wb_ref[...], preferred_element_type=jnp.float32)
    tr_ref[...] = jnp.concatenate([p, xx, pad], axis=1)
    tc_ref[...] = jnp.concatenate([q, xx, pad], axis=1)


def _pq_tables(hh, xx, we1a, we1b, blk):
    nseg, hdim = hh.shape
    grid = nseg // blk
    return pl.pallas_call(
        _pq_body,
        grid=(grid,),
        in_specs=[
            pl.BlockSpec((blk, hdim), lambda i: (i, 0)),
            pl.BlockSpec((blk, 3), lambda i: (i, 0)),
            pl.BlockSpec((hdim, hdim), lambda i: (0, 0)),
            pl.BlockSpec((hdim, hdim), lambda i: (0, 0)),
        ],
        out_specs=[
            pl.BlockSpec((blk, WIDE), lambda i: (i, 0)),
            pl.BlockSpec((blk, WIDE), lambda i: (i, 0)),
        ],
        out_shape=[
            jax.ShapeDtypeStruct((nseg, WIDE), jnp.float32),
            jax.ShapeDtypeStruct((nseg, WIDE), jnp.float32),
        ],
    )(hh, xx, we1a, we1b)


def _edge_body(gr_ref, gc_ref, ea_ref, wear_ref, wr_ref, be1_ref, we2_ref,
               be2_ref, wc1_ref, bc1_ref, wc2_ref, out_ref):
    hdim = we2_ref.shape[0]
    gr = gr_ref[...]
    gc = gc_ref[...]
    p = gr[:, :hdim]
    q = gc[:, :hdim]
    diff = gr[:, hdim:hdim + 3] - gc[:, hdim:hdim + 3]
    radial = jnp.sum(diff * diff, axis=1, keepdims=True)
    pre = p + q + radial * wr_ref[...]
    pre = pre + jnp.dot(ea_ref[...], wear_ref[...],
                        preferred_element_type=jnp.float32)
    pre = pre + be1_ref[...]
    m = _silu(pre)
    m = _silu(jnp.dot(m, we2_ref[...], preferred_element_type=jnp.float32)
              + be2_ref[...])
    c1 = _silu(jnp.dot(m, wc1_ref[...], preferred_element_type=jnp.float32)
               + bc1_ref[...])
    cw = jnp.dot(c1, wc2_ref[...], preferred_element_type=jnp.float32)
    b = m.shape[0]
    ones = jnp.ones((b, 1), jnp.float32)
    pad = jnp.zeros((b, WIDE - hdim - 4), jnp.float32)
    out_ref[...] = jnp.concatenate([m, diff * cw, ones, pad], axis=1)


def _edge_compute(gr, gc, ea, wear, wr, be1, we2, be2, wc1, bc1, wc2, blk):
    e = gr.shape[0]
    hdim = we2.shape[0]
    grid = e // blk
    full = lambda i: (0, 0)
    return pl.pallas_call(
        _edge_body,
        grid=(grid,),
        in_specs=[
            pl.BlockSpec((blk, WIDE), lambda i: (i, 0)),
            pl.BlockSpec((blk, WIDE), lambda i: (i, 0)),
            pl.BlockSpec((blk, 3), lambda i: (i, 0)),
            pl.BlockSpec((3, hdim), full),
            pl.BlockSpec((1, hdim), full),
            pl.BlockSpec((1, hdim), full),
            pl.BlockSpec((hdim, hdim), full),
            pl.BlockSpec((1, hdim), full),
            pl.BlockSpec((hdim, hdim), full),
            pl.BlockSpec((1, hdim), full),
            pl.BlockSpec((hdim, 1), full),
        ],
        out_specs=pl.BlockSpec((blk, WIDE), lambda i: (i, 0)),
        out_shape=jax.ShapeDtypeStruct((e, WIDE), jnp.float32),
    )(gr, gc, ea, wear, wr, be1, we2, be2, wc1, bc1, wc2)


def _node_body(hh_ref, tab_ref, vv_ref, xx_ref, wv_ref, bv_ref, wn1a_ref,
               wn1b_ref, bn1_ref, wn2_ref, bn2_ref, wea_ref, web_ref,
               hh_out, vv_out, xx_out, tr_out, tc_out):
    hdim = wv_ref.shape[0]
    hh = hh_ref[...]
    tab = tab_ref[...]
    magg = tab[:, :hdim]
    cnt = jnp.maximum(tab[:, hdim + 3:hdim + 4], 1.0)
    agg = tab[:, hdim:hdim + 3] / cnt
    wv = jnp.dot(hh, wv_ref[...], preferred_element_type=jnp.float32) + bv_ref[...]
    vvn = wv * vv_ref[...] + agg
    xxn = xx_ref[...] + vvn
    pre = (jnp.dot(hh, wn1a_ref[...], preferred_element_type=jnp.float32)
           + jnp.dot(magg, wn1b_ref[...], preferred_element_type=jnp.float32)
           + bn1_ref[...])
    hhn = jnp.dot(_silu(pre), wn2_ref[...],
                  preferred_element_type=jnp.float32) + bn2_ref[...]
    hh_out[...] = hhn
    vv_out[...] = vvn
    xx_out[...] = xxn
    b = hh.shape[0]
    pad = jnp.zeros((b, WIDE - hdim - 3), jnp.float32)
    p = jnp.dot(hhn, wea_ref[...], preferred_element_type=jnp.float32)
    q = jnp.dot(hhn, web_ref[...], preferred_element_type=jnp.float32)
    tr_out[...] = jnp.concatenate([p, xxn, pad], axis=1)
    tc_out[...] = jnp.concatenate([q, xxn, pad], axis=1)


def _node_update(hh, tab, vv, xx, wv, bv, wn1a, wn1b, bn1, wn2, bn2,
                 wea, web, blk):
    nseg, hdim = hh.shape
    grid = nseg // blk
    full = lambda i: (0, 0)
    return pl.pallas_call(
        _node_body,
        grid=(grid,),
        in_specs=[
            pl.BlockSpec((blk, hdim), lambda i: (i, 0)),
            pl.BlockSpec((blk, WIDE), lambda i: (i, 0)),
            pl.BlockSpec((blk, 3), lambda i: (i, 0)),
            pl.BlockSpec((blk, 3), lambda i: (i, 0)),
            pl.BlockSpec((hdim, 1), full),
            pl.BlockSpec((1, 1), full),
            pl.BlockSpec((hdim, hdim), full),
            pl.BlockSpec((hdim, hdim), full),
            pl.BlockSpec((1, hdim), full),
            pl.BlockSpec((hdim, hdim), full),
            pl.BlockSpec((1, hdim), full),
            pl.BlockSpec((hdim, hdim), full),
            pl.BlockSpec((hdim, hdim), full),
        ],
        out_specs=[
            pl.BlockSpec((blk, hdim), lambda i: (i, 0)),
            pl.BlockSpec((blk, 3), lambda i: (i, 0)),
            pl.BlockSpec((blk, 3), lambda i: (i, 0)),
            pl.BlockSpec((blk, WIDE), lambda i: (i, 0)),
            pl.BlockSpec((blk, WIDE), lambda i: (i, 0)),
        ],
        out_shape=[
            jax.ShapeDtypeStruct((nseg, hdim), jnp.float32),
            jax.ShapeDtypeStruct((nseg, 3), jnp.float32),
            jax.ShapeDtypeStruct((nseg, 3), jnp.float32),
            jax.ShapeDtypeStruct((nseg, WIDE), jnp.float32),
            jax.ShapeDtypeStruct((nseg, WIDE), jnp.float32),
        ],
    )(hh, tab, vv, xx, wv, bv, wn1a, wn1b, bn1, wn2, bn2, wea, web)


def _logit_body(hh_ref, wa_ref, ba_ref, out_ref):
    out_ref[...] = jax.nn.sigmoid(
        jnp.dot(hh_ref[...], wa_ref[...], preferred_element_type=jnp.float32)
        + ba_ref[...])


def _logits(hh, wa, ba, blk):
    nseg, hdim = hh.shape
    grid = nseg // blk
    return pl.pallas_call(
        _logit_body,
        grid=(grid,),
        in_specs=[
            pl.BlockSpec((blk, hdim), lambda i: (i, 0)),
            pl.BlockSpec((hdim, 1), lambda i: (0, 0)),
            pl.BlockSpec((1, 1), lambda i: (0, 0)),
        ],
        out_specs=pl.BlockSpec((blk, 1), lambda i: (i, 0)),
        out_shape=jax.ShapeDtypeStruct((nseg, 1), jnp.float32),
    )(hh, wa, ba)


def _pool_body(w_ref, x_ref, out_ref):
    w = w_ref[...]
    w = w / (jnp.sum(w, axis=1, keepdims=True) + 1e-8)
    out_ref[...] = jnp.sum(w[:, :, None] * x_ref[...], axis=1)


def _pool(w3, x3, blk):
    n, npast = w3.shape
    grid = n // blk
    return pl.pallas_call(
        _pool_body,
        grid=(grid,),
        in_specs=[
            pl.BlockSpec((blk, npast), lambda i: (i, 0)),
            pl.BlockSpec((blk, npast, 3), lambda i: (i, 0, 0)),
        ],
        out_specs=pl.BlockSpec((blk, 3), lambda i: (i, 0)),
        out_shape=jax.ShapeDtypeStruct((n, 3), jnp.float32),
    )(w3, x3)


def kernel(h, x, edges, edge_attr, node_vel, cfg, W_emb, b_emb, time_emb,
           We1, be1, We2, be2, Wc1, bc1, Wc2, Wn1, bn1, Wn2, bn2, Wv, bv,
           Wa, ba):
    npast, n = x.shape[0], x.shape[1]
    nseg = npast * n
    hdim = W_emb.shape[1]
    nlayers = We1.shape[0]
    e = edges.shape[1]

    n_blk = 1000 if n % 1000 == 0 else n
    seg_blk = 2000 if nseg % 2000 == 0 else nseg
    e_blk = 2000 if e % 2000 == 0 else e

    hh3 = _prologue(h, W_emb, b_emb.reshape(1, hdim), time_emb, n_blk)
    hh = hh3.reshape(nseg, hdim)
    xx = jnp.transpose(x, (1, 0, 2)).reshape(nseg, 3)
    vv = node_vel.reshape(nseg, 3)
    row, col = edges[0], edges[1]

    tr, tc = _pq_tables(hh, xx, We1[0, :hdim], We1[0, hdim:2 * hdim], seg_blk)

    sc_ok = (e % (_NC * _NS * _GCHUNK) == 0) and (e % (_NS * _SCH) == 0)
    ea_use = edge_attr
    offs8 = None
    if sc_ok:
        key = (row // _range_rows(nseg)).astype(jnp.int32)
        perm = jnp.argsort(key)
        row = jnp.take(row, perm)
        col = jnp.take(col, perm)
        ea_use = jnp.take(edge_attr, perm, axis=0)
        offs = jnp.searchsorted(
            jnp.take(key, perm), jnp.arange(_NB + 1, dtype=jnp.int32)
        ).astype(jnp.int32)
        offs8 = jnp.concatenate([offs, jnp.zeros((16 - _NB - 1,), jnp.int32)])
    for i in range(nlayers):
        if sc_ok:
            gr, gc = _sc_gather(tr, tc, row, col)
        else:
            gr = jnp.take(tr, row, axis=0)
            gc = jnp.take(tc, col, axis=0)
        eout = _edge_compute(
            gr, gc, ea_use,
            We1[i, 2 * hdim + 1:], We1[i, 2 * hdim:2 * hdim + 1],
            be1[i].reshape(1, hdim), We2[i], be2[i].reshape(1, hdim),
            Wc1[i], bc1[i].reshape(1, hdim), Wc2[i], e_blk)
        if sc_ok:
            tab4 = _sc_scatter(eout, row, offs8, nseg)
            rr = tab4.shape[1] - 32
            tab = jnp.concatenate(
                [tab4[b, :rr] for b in range(_NB)], axis=0)[:nseg]
        else:
            tab = jax.ops.segment_sum(eout, row, num_segments=nseg)
        if i + 1 < nlayers:
            wea = We1[i + 1, :hdim]
            web = We1[i + 1, hdim:2 * hdim]
        else:
            wea = jnp.zeros((hdim, hdim), jnp.float32)
            web = jnp.zeros((hdim, hdim), jnp.float32)
        hh, vv, xx, tr, tc = _node_update(
            hh, tab, vv, xx, Wv[i], bv[i].reshape(1, 1),
            Wn1[i, :hdim], Wn1[i, hdim:], bn1[i].reshape(1, hdim),
            Wn2[i], bn2[i].reshape(1, hdim), wea, web, seg_blk)

    w = _logits(hh, Wa, ba.reshape(1, 1), seg_blk)
    w3 = w.reshape(n, npast)
    x3 = xx.reshape(n, npast, 3)
    return _pool(w3, x3, n_blk)
